# baseline (speedup 1.0000x reference)
import jax
import jax.numpy as jnp
from jax.experimental import pallas as pl

_B = 128
_N = 8192
_BR = 8


def _onehot_body(x_ref, o_ref):
    x = x_ref[...]
    m = jnp.max(x, axis=1, keepdims=True)
    iota = jax.lax.broadcasted_iota(jnp.int32, x.shape, 1)
    cand = jnp.where(x == m, iota, _N)
    idx = jnp.min(cand, axis=1, keepdims=True)
    o_ref[...] = (iota == idx).astype(jnp.float32)


def kernel(coords):
    return pl.pallas_call(
        _onehot_body,
        out_shape=jax.ShapeDtypeStruct((_B, _N), jnp.float32),
        grid=(_B // _BR,),
        in_specs=[pl.BlockSpec((_BR, _N), lambda i: (i, 0))],
        out_specs=pl.BlockSpec((_BR, _N), lambda i: (i, 0)),
    )(coords)


# speedup vs baseline: 2.1556x; 2.1556x over previous
import jax
import jax.numpy as jnp
from jax.experimental import pallas as pl

_B = 128
_N = 8192
_BR = 32


def _onehot_body(x_ref, o_ref):
    x = x_ref[...]
    m = jnp.max(x, axis=1, keepdims=True)
    iota = jax.lax.broadcasted_iota(jnp.int32, x.shape, 1)
    cand = jnp.where(x == m, iota, _N)
    idx = jnp.min(cand, axis=1, keepdims=True)
    o_ref[...] = (iota == idx).astype(jnp.float32)


def kernel(coords):
    return pl.pallas_call(
        _onehot_body,
        out_shape=jax.ShapeDtypeStruct((_B, _N), jnp.float32),
        grid=(_B // _BR,),
        in_specs=[pl.BlockSpec((_BR, _N), lambda i: (i, 0))],
        out_specs=pl.BlockSpec((_BR, _N), lambda i: (i, 0)),
    )(coords)


# speedup vs baseline: 2.8068x; 1.3021x over previous
import jax
import jax.numpy as jnp
from jax.experimental import pallas as pl

_B = 128
_N = 8192
_BR = 64


def _onehot_body(x_ref, o_ref):
    x = x_ref[...]
    m = jnp.max(x, axis=1, keepdims=True)
    iota = jax.lax.broadcasted_iota(jnp.int32, x.shape, 1)
    cand = jnp.where(x == m, iota, _N)
    idx = jnp.min(cand, axis=1, keepdims=True)
    o_ref[...] = (iota == idx).astype(jnp.float32)


def kernel(coords):
    return pl.pallas_call(
        _onehot_body,
        out_shape=jax.ShapeDtypeStruct((_B, _N), jnp.float32),
        grid=(_B // _BR,),
        in_specs=[pl.BlockSpec((_BR, _N), lambda i: (i, 0))],
        out_specs=pl.BlockSpec((_BR, _N), lambda i: (i, 0)),
    )(coords)
